# baseline (speedup 1.0000x reference)
import functools

import jax
import jax.numpy as jnp
from jax import lax
from jax.experimental import pallas as pl
from jax.experimental.pallas import tpu as pltpu
from jax.experimental.pallas import tpu_sc as plsc

B, N, D, R = 128, 1024, 384, 512
TOTAL = B * N
V = R * R

_info = plsc.get_sparse_core_info()
NC, NS, L = _info.num_cores, _info.num_subcores, _info.num_lanes
NW = NC * NS
W = TOTAL // NW
C = 32
NCHUNK = W // C
NBUF = 4
K = NBUF - 1


def _sc_gather_add(x2, xi, yi, table2):
    mesh = plsc.VectorSubcoreMesh(core_axis_name="c", subcore_axis_name="s")

    @functools.partial(
        pl.kernel,
        mesh=mesh,
        out_type=jax.ShapeDtypeStruct((TOTAL, D), jnp.float32),
        scratch_types=(
            [pltpu.VMEM((W,), jnp.int32)] * 3
            + [pltpu.VMEM((C, D), jnp.float32)] * NBUF
            + [pltpu.VMEM((C, D), jnp.float32)] * NBUF
            + [pltpu.SemaphoreType.DMA] * (3 * NBUF)
        ),
    )
    def k(x_hbm, xi_hbm, yi_hbm, tab_hbm, out_hbm, xi_v, yi_v, idx_v, *bufs):
        xbufs = bufs[0:NBUF]
        rbufs = bufs[NBUF:2 * NBUF]
        in_s = bufs[2 * NBUF:3 * NBUF]
        ga_s = bufs[3 * NBUF:4 * NBUF]
        out_s = bufs[4 * NBUF:5 * NBUF]
        wid = lax.axis_index("s") * NC + lax.axis_index("c")
        base = wid * W

        def start_in(c, b):
            return pltpu.async_copy(x_hbm.at[pl.ds(base + c * C, C)], xbufs[b], in_s[b])

        def start_ga(c, b):
            return pltpu.async_copy(
                tab_hbm.at[idx_v.at[pl.ds(c * C, C)]], rbufs[b], ga_s[b])

        def wait_in(c, b):
            pltpu.make_async_copy(
                x_hbm.at[pl.ds(base + c * C, C)], xbufs[b], in_s[b]).wait()

        def wait_ga(c, b):
            pltpu.make_async_copy(
                tab_hbm.at[idx_v.at[pl.ds(c * C, C)]], rbufs[b], ga_s[b]).wait()

        def start_out(c, b):
            return pltpu.async_copy(
                xbufs[b], out_hbm.at[pl.ds(base + c * C, C)], out_s[b])

        def wait_out(c, b):
            pltpu.make_async_copy(
                xbufs[b], out_hbm.at[pl.ds(base + c * C, C)], out_s[b]).wait()

        def add_chunk(b):
            xb, rb = xbufs[b], rbufs[b]

            def add_row(r, carry):
                for j in range(D // L):
                    s = pl.ds(j * L, L)
                    plsc.addupdate(xb.at[r, s], rb[r, s])
                return carry

            lax.fori_loop(0, C, add_row, 0)

        for c0 in range(K):
            start_in(c0, c0)
        pltpu.sync_copy(xi_hbm.at[pl.ds(base, W)], xi_v)
        pltpu.sync_copy(yi_hbm.at[pl.ds(base, W)], yi_v)

        def compute_idx(t, carry):
            s = pl.ds(t * L, L)
            idx_v[s] = yi_v[s] * R + xi_v[s]
            return carry

        lax.fori_loop(0, W // L, compute_idx, 0)
        for c0 in range(K):
            start_ga(c0, c0)

        wait_in(0, 0)
        wait_ga(0, 0)
        add_chunk(0)
        start_out(0, 0)
        start_in(K, K % NBUF)
        start_ga(K, K % NBUF)

        def quad_body(q, carry):
            for j in range(NBUF):
                c = NBUF * q + 1 + j
                b = (1 + j) % NBUF
                wait_in(c, b)
                wait_ga(c, b)
                add_chunk(b)
                start_out(c, b)
                bp = (b + K) % NBUF
                wait_out(c - 1, bp)
                start_in(c + K, bp)
                start_ga(c + K, bp)
            return carry

        lax.fori_loop(0, (NCHUNK - NBUF) // NBUF, quad_body, 0)

        for c in range(NCHUNK - K, NCHUNK):
            b = c % NBUF
            wait_in(c, b)
            wait_ga(c, b)
            add_chunk(b)
            start_out(c, b)
        for c in range(NCHUNK - NBUF, NCHUNK):
            wait_out(c, c % NBUF)

    return k(x2, xi, yi, table2)


def kernel(x, offgrid_coords, pos_table):
    x2 = x.reshape(TOTAL, D)
    xi = offgrid_coords[..., 0].reshape(TOTAL)
    yi = offgrid_coords[..., 1].reshape(TOTAL)
    table2 = pos_table.reshape(V, D)
    out = _sc_gather_add(x2, xi, yi, table2)
    return out.reshape(B, N, D)
